# Initial kernel scaffold; baseline (speedup 1.0000x reference)
import jax
import jax.numpy as jnp
from jax.experimental import pallas as pl


def kernel(x, A, W, temperature):
    raise NotImplementedError("write your pallas kernel here")


# speedup vs baseline: 5.3048x; 5.3048x over previous
import functools

import jax
import jax.numpy as jnp
from jax.experimental import pallas as pl
from jax.experimental.pallas import tpu as pltpu

_K = 16
_ROT_A = (13, 15, 26, 6)
_ROT_B = (17, 29, 16, 24)
_KS = (0, 1, 0x1BD11BDA ^ 0 ^ 1)


def _threefry_bits(x1):
    ks = [jnp.uint32(k) for k in _KS]
    x0 = jnp.zeros_like(x1) + ks[0]
    x1 = x1 + ks[1]

    def rotl(v, r):
        return (v << jnp.uint32(r)) | (v >> jnp.uint32(32 - r))

    for i in range(5):
        rots = _ROT_A if i % 2 == 0 else _ROT_B
        for r in rots:
            x0 = x0 + x1
            x1 = rotl(x1, r)
            x1 = x1 ^ x0
        x0 = x0 + ks[(i + 1) % 3]
        x1 = x1 + ks[(i + 2) % 3] + jnp.uint32(i + 1)
    return x0 ^ x1


def _embed_kernel(x_ref, w_ref, xx_ref):
    xv = jnp.clip(x_ref[...], -1000000.0, 1000000.0)
    y = jnp.dot(xv, w_ref[...], preferred_element_type=jnp.float32,
                precision=jax.lax.Precision.DEFAULT)
    xx_ref[...] = jnp.clip(y, -1000000.0, 1000000.0)


def _dist_topk_kernel(temp_ref, rows_ref, all_ref, lp_ref, idx_ref, *, n, tile_r, k):
    b = pl.program_id(0)
    i = pl.program_id(1)
    rows = rows_ref[0]
    alln = all_ref[0]
    sq_r = jnp.sum(rows * rows, axis=1, keepdims=True)
    sq_c = jnp.sum(alln * alln, axis=1)[None, :]
    gram = jax.lax.dot_general(
        rows, alln, (((1,), (1,)), ((), ())),
        preferred_element_type=jnp.float32,
        precision=jax.lax.Precision.DEFAULT)
    dm = sq_r + sq_c - 2.0 * gram
    dm = jnp.maximum(dm, 0.0)
    dm = jnp.where(jnp.isnan(dm) | jnp.isinf(dm), 1000000.0, dm)
    logits = jnp.clip(dm, -1e10, 1e10) * temp_ref[0]

    r_iota = jax.lax.broadcasted_iota(jnp.uint32, (tile_r, n), 0)
    c_iota = jax.lax.broadcasted_iota(jnp.uint32, (tile_r, n), 1)
    base = (b.astype(jnp.uint32) * jnp.uint32(n) +
            i.astype(jnp.uint32) * jnp.uint32(tile_r)) * jnp.uint32(n)
    lin = base + r_iota * jnp.uint32(n) + c_iota
    bits = _threefry_bits(lin)
    fb = (bits >> jnp.uint32(9)) | jnp.uint32(0x3F800000)
    u = jax.lax.bitcast_convert_type(fb, jnp.float32) - 1.0
    q = jnp.clip(u, 1e-10, 1.0 - 1e-10)
    lnlq = jnp.log(-jnp.log(q))
    lq = logits - lnlq
    lq = jnp.where(jnp.isnan(lq) | jnp.isinf(lq), -1e10, lq)

    col = jax.lax.broadcasted_iota(jnp.int32, (tile_r, n), 1)
    work = lq
    vals, idxs = [], []
    for _ in range(k):
        m = jnp.min(work, axis=1, keepdims=True)
        j = jnp.min(jnp.where(work == m, col, n), axis=1, keepdims=True)
        vals.append(m)
        idxs.append(j)
        work = jnp.where(col == j, jnp.float32(jnp.inf), work)
    lp_ref[0] = jnp.clip(jnp.concatenate(vals, axis=1), -1e10, 0.0)
    idx_ref[0] = jnp.concatenate(idxs, axis=1)


def kernel(x, A, W, temperature):
    b, n, d = x.shape
    k = _K
    x_flat = x.reshape(b * n, d)
    tile_m = 1024
    xx_flat = pl.pallas_call(
        _embed_kernel,
        grid=(b * n // tile_m,),
        in_specs=[
            pl.BlockSpec((tile_m, d), lambda m: (m, 0)),
            pl.BlockSpec((d, d), lambda m: (0, 0)),
        ],
        out_specs=pl.BlockSpec((tile_m, d), lambda m: (m, 0)),
        out_shape=jax.ShapeDtypeStruct((b * n, d), jnp.float32),
    )(x_flat, W)
    xx = xx_flat.reshape(b, n, d)

    temp = jnp.exp(jnp.clip(temperature, -5.0, 5.0)).reshape(1)

    tile_r = 256
    lp, idx = pl.pallas_call(
        functools.partial(_dist_topk_kernel, n=n, tile_r=tile_r, k=k),
        grid=(b, n // tile_r),
        in_specs=[
            pl.BlockSpec(memory_space=pltpu.SMEM),
            pl.BlockSpec((1, tile_r, d), lambda bb, i: (bb, i, 0)),
            pl.BlockSpec((1, n, d), lambda bb, i: (bb, 0, 0)),
        ],
        out_specs=[
            pl.BlockSpec((1, tile_r, k), lambda bb, i: (bb, i, 0)),
            pl.BlockSpec((1, tile_r, k), lambda bb, i: (bb, i, 0)),
        ],
        out_shape=[
            jax.ShapeDtypeStruct((b, n, k), jnp.float32),
            jax.ShapeDtypeStruct((b, n, k), jnp.int32),
        ],
    )(temp, xx, xx)

    off = (jnp.arange(b, dtype=jnp.int32) * n)[:, None, None]
    src = jnp.broadcast_to(jnp.arange(n, dtype=jnp.int32)[None, :, None], (b, n, k))
    edges = jnp.stack([(src + off).reshape(-1), (idx + off).reshape(-1)], axis=0)
    return xx, edges, lp


# speedup vs baseline: 6.0963x; 1.1492x over previous
import functools

import jax
import jax.numpy as jnp
from jax.experimental import pallas as pl
from jax.experimental.pallas import tpu as pltpu

_K = 16
_ROT_A = (13, 15, 26, 6)
_ROT_B = (17, 29, 16, 24)
_KS = (0, 1, 0x1BD11BDA ^ 0 ^ 1)


def _threefry_bits(x1):
    ks = [jnp.uint32(k) for k in _KS]
    x0 = jnp.zeros_like(x1) + ks[0]
    x1 = x1 + ks[1]

    def rotl(v, r):
        return (v << jnp.uint32(r)) | (v >> jnp.uint32(32 - r))

    for i in range(5):
        rots = _ROT_A if i % 2 == 0 else _ROT_B
        for r in rots:
            x0 = x0 + x1
            x1 = rotl(x1, r)
            x1 = x1 ^ x0
        x0 = x0 + ks[(i + 1) % 3]
        x1 = x1 + ks[(i + 2) % 3] + jnp.uint32(i + 1)
    return x0 ^ x1


def _embed_kernel(x_ref, w_ref, xx_ref):
    xv = jnp.clip(x_ref[...], -1000000.0, 1000000.0)
    y = jnp.dot(xv, w_ref[...], preferred_element_type=jnp.float32,
                precision=jax.lax.Precision.DEFAULT)
    xx_ref[...] = jnp.clip(y, -1000000.0, 1000000.0)


def _dist_topk_kernel(temp_ref, rows_ref, all_ref, lp_ref, idx_ref, *, n, tile_r, k):
    b = pl.program_id(0)
    i = pl.program_id(1)
    rows = rows_ref[0]
    alln = all_ref[0]
    sq_r = jnp.sum(rows * rows, axis=1, keepdims=True)
    sq_c = jnp.sum(alln * alln, axis=1)[None, :]
    gram = jax.lax.dot_general(
        rows, alln, (((1,), (1,)), ((), ())),
        preferred_element_type=jnp.float32,
        precision=jax.lax.Precision.DEFAULT)
    dm = sq_r + sq_c - 2.0 * gram
    dm = jnp.maximum(dm, 0.0)
    logits = jnp.clip(dm, -1e10, 1e10) * temp_ref[0]

    r_iota = jax.lax.broadcasted_iota(jnp.uint32, (tile_r, n), 0)
    c_iota = jax.lax.broadcasted_iota(jnp.uint32, (tile_r, n), 1)
    base = (b.astype(jnp.uint32) * jnp.uint32(n) +
            i.astype(jnp.uint32) * jnp.uint32(tile_r)) * jnp.uint32(n)
    lin = base + r_iota * jnp.uint32(n) + c_iota
    bits = _threefry_bits(lin)
    fb = (bits >> jnp.uint32(9)) | jnp.uint32(0x3F800000)
    u = jax.lax.bitcast_convert_type(fb, jnp.float32) - 1.0
    q = jnp.clip(u, 1e-10, 1.0 - 1e-10)
    lnlq = jnp.log(-jnp.log(q))
    lq = logits - lnlq

    col = jax.lax.broadcasted_iota(jnp.int32, (tile_r, n), 1)
    work = lq
    vals, idxs = [], []
    for _ in range(k):
        m = jnp.min(work, axis=1, keepdims=True)
        hit = work == m
        j = jnp.min(jnp.where(hit, col, n), axis=1, keepdims=True)
        vals.append(m)
        idxs.append(j)
        work = jnp.where(hit, jnp.float32(jnp.inf), work)
    lp_ref[0] = jnp.clip(jnp.concatenate(vals, axis=1), -1e10, 0.0)
    idx_ref[0] = jnp.concatenate(idxs, axis=1)


def kernel(x, A, W, temperature):
    b, n, d = x.shape
    k = _K
    x_flat = x.reshape(b * n, d)
    tile_m = 1024
    xx_flat = pl.pallas_call(
        _embed_kernel,
        grid=(b * n // tile_m,),
        in_specs=[
            pl.BlockSpec((tile_m, d), lambda m: (m, 0)),
            pl.BlockSpec((d, d), lambda m: (0, 0)),
        ],
        out_specs=pl.BlockSpec((tile_m, d), lambda m: (m, 0)),
        out_shape=jax.ShapeDtypeStruct((b * n, d), jnp.float32),
    )(x_flat, W)
    xx = xx_flat.reshape(b, n, d)

    temp = jnp.exp(jnp.clip(temperature, -5.0, 5.0)).reshape(1)

    tile_r = 256
    lp, idx = pl.pallas_call(
        functools.partial(_dist_topk_kernel, n=n, tile_r=tile_r, k=k),
        grid=(b, n // tile_r),
        in_specs=[
            pl.BlockSpec(memory_space=pltpu.SMEM),
            pl.BlockSpec((1, tile_r, d), lambda bb, i: (bb, i, 0)),
            pl.BlockSpec((1, n, d), lambda bb, i: (bb, 0, 0)),
        ],
        out_specs=[
            pl.BlockSpec((1, tile_r, k), lambda bb, i: (bb, i, 0)),
            pl.BlockSpec((1, tile_r, k), lambda bb, i: (bb, i, 0)),
        ],
        out_shape=[
            jax.ShapeDtypeStruct((b, n, k), jnp.float32),
            jax.ShapeDtypeStruct((b, n, k), jnp.int32),
        ],
        compiler_params=pltpu.CompilerParams(
            dimension_semantics=("parallel", "parallel")),
    )(temp, xx, xx)

    off = (jnp.arange(b, dtype=jnp.int32) * n)[:, None, None]
    src = jnp.broadcast_to(jnp.arange(n, dtype=jnp.int32)[None, :, None], (b, n, k))
    edges = jnp.stack([(src + off).reshape(-1), (idx + off).reshape(-1)], axis=0)
    return xx, edges, lp


# speedup vs baseline: 6.6293x; 1.0874x over previous
import functools

import jax
import jax.numpy as jnp
from jax.experimental import pallas as pl
from jax.experimental.pallas import tpu as pltpu

_K = 16
_ROT_A = (13, 15, 26, 6)
_ROT_B = (17, 29, 16, 24)
_KS = (0, 1, 0x1BD11BDA ^ 0 ^ 1)


def _threefry_bits(x1):
    ks = [jnp.uint32(k) for k in _KS]
    x0 = jnp.zeros_like(x1) + ks[0]
    x1 = x1 + ks[1]

    def rotl(v, r):
        return (v << jnp.uint32(r)) | (v >> jnp.uint32(32 - r))

    for i in range(5):
        rots = _ROT_A if i % 2 == 0 else _ROT_B
        for r in rots:
            x0 = x0 + x1
            x1 = rotl(x1, r)
            x1 = x1 ^ x0
        x0 = x0 + ks[(i + 1) % 3]
        x1 = x1 + ks[(i + 2) % 3] + jnp.uint32(i + 1)
    return x0 ^ x1


def _embed_kernel(x_ref, w_ref, xx_ref):
    xv = jnp.clip(x_ref[...], -1000000.0, 1000000.0)
    y = jnp.dot(xv, w_ref[...], preferred_element_type=jnp.float32,
                precision=jax.lax.Precision.DEFAULT)
    xx_ref[...] = jnp.clip(y, -1000000.0, 1000000.0)


def _dist_topk_kernel(temp_ref, rows_ref, all_ref, lp_ref, idx_ref, *, n, tile_r, k):
    b = pl.program_id(0)
    i = pl.program_id(1)
    rows = rows_ref[0]
    alln = all_ref[0]
    sq_r = jnp.sum(rows * rows, axis=1, keepdims=True)
    sq_c = jnp.sum(alln * alln, axis=1)[None, :]
    gram = jax.lax.dot_general(
        rows, alln, (((1,), (1,)), ((), ())),
        preferred_element_type=jnp.float32,
        precision=jax.lax.Precision.DEFAULT)
    dm = sq_r + sq_c - 2.0 * gram
    dm = jnp.maximum(dm, 0.0)
    logits = jnp.clip(dm, -1e10, 1e10) * temp_ref[0]

    r_iota = jax.lax.broadcasted_iota(jnp.uint32, (tile_r, n), 0)
    c_iota = jax.lax.broadcasted_iota(jnp.uint32, (tile_r, n), 1)
    base = (b.astype(jnp.uint32) * jnp.uint32(n) +
            i.astype(jnp.uint32) * jnp.uint32(tile_r)) * jnp.uint32(n)
    lin = base + r_iota * jnp.uint32(n) + c_iota
    bits = _threefry_bits(lin)
    fb = (bits >> jnp.uint32(9)) | jnp.uint32(0x3F800000)
    u = jax.lax.bitcast_convert_type(fb, jnp.float32) - 1.0
    q = jnp.clip(u, 1e-10, 1.0 - 1e-10)
    lnlq = jnp.log(-jnp.log(q))
    lq = logits - lnlq

    colf = jax.lax.broadcasted_iota(jnp.int32, (tile_r, n), 1).astype(jnp.float32)
    work = lq
    vals, idxs = [], []
    for _ in range(k):
        m = jnp.min(work, axis=1, keepdims=True)
        hit = work == m
        j = jnp.min(jnp.where(hit, colf, jnp.float32(n)), axis=1, keepdims=True)
        vals.append(m)
        idxs.append(j)
        work = jnp.where(hit, jnp.float32(jnp.inf), work)
    lp_ref[0] = jnp.clip(jnp.concatenate(vals, axis=1), -1e10, 0.0)
    idx_ref[0] = jnp.concatenate(idxs, axis=1).astype(jnp.int32)


def kernel(x, A, W, temperature):
    b, n, d = x.shape
    k = _K
    x_flat = x.reshape(b * n, d)
    tile_m = 1024
    xx_flat = pl.pallas_call(
        _embed_kernel,
        grid=(b * n // tile_m,),
        in_specs=[
            pl.BlockSpec((tile_m, d), lambda m: (m, 0)),
            pl.BlockSpec((d, d), lambda m: (0, 0)),
        ],
        out_specs=pl.BlockSpec((tile_m, d), lambda m: (m, 0)),
        out_shape=jax.ShapeDtypeStruct((b * n, d), jnp.float32),
    )(x_flat, W)
    xx = xx_flat.reshape(b, n, d)

    temp = jnp.exp(jnp.clip(temperature, -5.0, 5.0)).reshape(1)

    tile_r = 256
    lp, idx = pl.pallas_call(
        functools.partial(_dist_topk_kernel, n=n, tile_r=tile_r, k=k),
        grid=(b, n // tile_r),
        in_specs=[
            pl.BlockSpec(memory_space=pltpu.SMEM),
            pl.BlockSpec((1, tile_r, d), lambda bb, i: (bb, i, 0)),
            pl.BlockSpec((1, n, d), lambda bb, i: (bb, 0, 0)),
        ],
        out_specs=[
            pl.BlockSpec((1, tile_r, k), lambda bb, i: (bb, i, 0)),
            pl.BlockSpec((1, tile_r, k), lambda bb, i: (bb, i, 0)),
        ],
        out_shape=[
            jax.ShapeDtypeStruct((b, n, k), jnp.float32),
            jax.ShapeDtypeStruct((b, n, k), jnp.int32),
        ],
        compiler_params=pltpu.CompilerParams(
            dimension_semantics=("parallel", "parallel")),
    )(temp, xx, xx)

    off = (jnp.arange(b, dtype=jnp.int32) * n)[:, None, None]
    src = jnp.broadcast_to(jnp.arange(n, dtype=jnp.int32)[None, :, None], (b, n, k))
    edges = jnp.stack([(src + off).reshape(-1), (idx + off).reshape(-1)], axis=0)
    return xx, edges, lp
